# Initial kernel scaffold; baseline (speedup 1.0000x reference)
import jax
import jax.numpy as jnp
from jax.experimental import pallas as pl


def kernel(offset_map_pred, hm_gt, offset_gt):
    raise NotImplementedError("write your pallas kernel here")


# speedup vs baseline: 1.2766x; 1.2766x over previous
import functools

import jax
import jax.numpy as jnp
from jax import lax
from jax.experimental import pallas as pl
from jax.experimental.pallas import tpu as pltpu

_B = 32
_N = 17
_HW = 128 * 128
_C = 128
_NCHUNK = _HW // _C


def _loss_kernel(hm_ref, off_ref, gt_ref, out_ref):
    i = pl.program_id(0)

    def body(f, carry):
        run_max, run_f, run_ox, run_oy = carry
        sl = pl.ds(f * _C, _C)
        hm_c = hm_ref[0, :, sl]
        ox_c = off_ref[0, 0, sl]
        oy_c = off_ref[0, 1, sl]
        upd = hm_c > run_max
        run_max = jnp.where(upd, hm_c, run_max)
        run_f = jnp.where(upd, f, run_f)
        run_ox = jnp.where(upd, ox_c, run_ox)
        run_oy = jnp.where(upd, oy_c, run_oy)
        return run_max, run_f, run_ox, run_oy

    init = (
        jnp.full((_N, _C), -jnp.inf, jnp.float32),
        jnp.zeros((_N, _C), jnp.int32),
        jnp.zeros((_N, _C), jnp.float32),
        jnp.zeros((_N, _C), jnp.float32),
    )
    run_max, run_f, run_ox, run_oy = lax.fori_loop(
        0, _NCHUNK, body, init, unroll=4
    )

    m = jnp.max(run_max, axis=-1, keepdims=True)
    lane_iota = lax.broadcasted_iota(jnp.int32, (_N, _C), 1)
    flat = run_f * _C + lane_iota
    masked_flat = jnp.where(run_max == m, flat, jnp.int32(_HW))
    win_flat = jnp.min(masked_flat, axis=-1, keepdims=True)
    win = masked_flat == win_flat
    ox = jnp.sum(jnp.where(win, run_ox, 0.0), axis=-1)
    oy = jnp.sum(jnp.where(win, run_oy, 0.0), axis=-1)

    gt = gt_ref[0]
    partial = jnp.sum(jnp.abs(ox - gt[:, 0]) + jnp.abs(oy - gt[:, 1]))

    @pl.when(i == 0)
    def _init():
        out_ref[0] = 0.0

    out_ref[0] += partial

    @pl.when(i == _B - 1)
    def _finish():
        out_ref[0] = out_ref[0] * (1.0 / (_B * _N * 2 * _N))


@functools.partial(jax.jit)
def _run(hm_flat, off_flat, offset_gt):
    out = pl.pallas_call(
        _loss_kernel,
        grid=(_B,),
        in_specs=[
            pl.BlockSpec((1, _N, _HW), lambda i: (i, 0, 0)),
            pl.BlockSpec((1, 2, _HW), lambda i: (i, 0, 0)),
            pl.BlockSpec((1, _N, 2), lambda i: (i, 0, 0)),
        ],
        out_specs=pl.BlockSpec(memory_space=pltpu.MemorySpace.SMEM),
        out_shape=jax.ShapeDtypeStruct((1,), jnp.float32),
    )(hm_flat, off_flat, offset_gt)
    return out[0]


def kernel(offset_map_pred, hm_gt, offset_gt):
    b, n = hm_gt.shape[0], hm_gt.shape[1]
    hm_flat = hm_gt.reshape(b, n, -1)
    off_flat = offset_map_pred.reshape(b, 2, -1)
    return _run(hm_flat, off_flat, offset_gt)


# speedup vs baseline: 2.2980x; 1.8002x over previous
import functools

import jax
import jax.numpy as jnp
from jax import lax
from jax.experimental import pallas as pl
from jax.experimental.pallas import tpu as pltpu
from jax.experimental.pallas import tpu_sc as plsc

_B = 32
_N = 17
_HW = 16384
_L = 16
_NCHUNK = _HW // _L
_GTP = 48


def _make_sc_call():
    mesh = plsc.VectorSubcoreMesh(core_axis_name="c", subcore_axis_name="s")

    @functools.partial(
        pl.kernel,
        mesh=mesh,
        out_type=jax.ShapeDtypeStruct((_B * _L,), jnp.float32),
        scratch_types=[
            pltpu.VMEM((_HW,), jnp.float32),
            pltpu.VMEM((_HW,), jnp.float32),
            pltpu.VMEM((2 * _HW + _L,), jnp.float32),
            pltpu.VMEM((_GTP,), jnp.float32),
            pltpu.VMEM((_L,), jnp.float32),
            pltpu.SemaphoreType.DMA,
            pltpu.SemaphoreType.DMA,
            pltpu.SemaphoreType.DMA,
        ],
    )
    def sc_loss(hm_hbm, off_hbm, gt_hbm, out_hbm,
                row_a, row_b, off_v, gt_v, part_v,
                sem_a, sem_b, sem_c):
        w = lax.axis_index("s") * 2 + lax.axis_index("c")

        off_cp = pltpu.async_copy(
            off_hbm.at[pl.ds(w * (2 * _HW), 2 * _HW)],
            off_v.at[pl.ds(0, 2 * _HW)],
            sem_c,
        )
        pltpu.sync_copy(gt_hbm.at[pl.ds(w * _GTP, _GTP)], gt_v)

        hm_base = w * (_N * _HW)
        bufs = (row_a, row_b)
        sems = (sem_a, sem_b)
        copies = [None, None]
        copies[0] = pltpu.async_copy(
            hm_hbm.at[pl.ds(hm_base, _HW)], row_a, sems[0]
        )

        lane = lax.broadcasted_iota(jnp.int32, (_L,), 0)
        ox = jnp.zeros((_L,), jnp.float32)
        oy = jnp.zeros((_L,), jnp.float32)
        gx = jnp.zeros((_L,), jnp.float32)
        gy = jnp.zeros((_L,), jnp.float32)
        off_waited = False

        for k in range(_N):
            buf = bufs[k % 2]
            copies[k % 2].wait()
            if k + 1 < _N:
                copies[(k + 1) % 2] = pltpu.async_copy(
                    hm_hbm.at[pl.ds(hm_base + (k + 1) * _HW, _HW)],
                    bufs[(k + 1) % 2],
                    sems[(k + 1) % 2],
                )

            def chunk_body(j, carry, buf=buf):
                run_max, run_j = carry
                v = buf[pl.ds(j * _L, _L)]
                upd = v > run_max
                return (
                    jnp.where(upd, v, run_max),
                    jnp.where(upd, j, run_j),
                )

            run_max, run_j = lax.fori_loop(
                0,
                _NCHUNK,
                chunk_body,
                (
                    jnp.full((_L,), -jnp.inf, jnp.float32),
                    jnp.zeros((_L,), jnp.int32),
                ),
                unroll=8,
            )

            best_v = run_max
            best_f = run_j * _L + lane
            for s in (8, 4, 2, 1):
                perm = lane ^ s
                o_v = best_v.at[perm].get(mode="promise_in_bounds")
                o_f = best_f.at[perm].get(mode="promise_in_bounds")
                upd = (o_v > best_v) | ((o_v == best_v) & (o_f < best_f))
                best_v = jnp.where(upd, o_v, best_v)
                best_f = jnp.where(upd, o_f, best_f)

            idx_k = best_f[0]

            if not off_waited:
                off_cp.wait()
                off_waited = True
            ox_k = off_v[pl.ds(idx_k, _L)][0]
            oy_k = off_v[pl.ds(idx_k + _HW, _L)][0]
            gvec = gt_v[pl.ds(2 * k, _L)]
            gx_k = gvec[0]
            gy_k = gvec[1]
            tgt = k % _L
            ox = jnp.where(lane == tgt, ox_k, ox) if k < _L else ox
            oy = jnp.where(lane == tgt, oy_k, oy) if k < _L else oy
            gx = jnp.where(lane == tgt, gx_k, gx) if k < _L else gx
            gy = jnp.where(lane == tgt, gy_k, gy) if k < _L else gy
            if k >= _L:
                extra = (k, ox_k, oy_k, gx_k, gy_k)

        err = jnp.abs(ox - gx) + jnp.abs(oy - gy)
        _, eox, eoy, egx, egy = extra
        err_extra = jnp.abs(eox - egx) + jnp.abs(eoy - egy)
        err = err + jnp.where(lane == 0, err_extra, 0.0)
        part_v[...] = err
        pltpu.sync_copy(part_v, out_hbm.at[pl.ds(w * _L, _L)])

    return sc_loss


_sc_call = _make_sc_call()


@jax.jit
def _run(hm_flat, off_flat, gt_pad):
    parts = _sc_call(hm_flat, off_flat, gt_pad)
    return jnp.sum(parts) * (1.0 / (_B * _N * 2 * _N))


def kernel(offset_map_pred, hm_gt, offset_gt):
    b, n = hm_gt.shape[0], hm_gt.shape[1]
    hm_flat = hm_gt.reshape(-1)
    off_flat = offset_map_pred.reshape(-1)
    gt_pad = jnp.zeros((b, _GTP), jnp.float32)
    gt_pad = gt_pad.at[:, : 2 * n].set(offset_gt.reshape(b, 2 * n))
    return _run(hm_flat, off_flat, gt_pad.reshape(-1))
